# Initial kernel scaffold; baseline (speedup 1.0000x reference)
import functools

import jax
import jax.numpy as jnp
from jax import lax
from jax.experimental import pallas as pl
from jax.experimental.pallas import tpu as pltpu
from jax.experimental.pallas import tpu_sc as plsc

B = 1
S = 2048
D = 1024
F = 1024
E = 8
V = 4096
N_TOK = B * S
TILE = 256
N_TILES = N_TOK // TILE


@functools.lru_cache(maxsize=None)
def _sc_gather_fn():
    info = plsc.get_sparse_core_info()
    nw = info.num_cores * info.num_subcores
    b_per_w = N_TOK // nw
    mesh = plsc.VectorSubcoreMesh(core_axis_name="c", subcore_axis_name="s")

    @functools.partial(
        pl.kernel,
        mesh=mesh,
        out_type=jax.ShapeDtypeStruct((N_TOK, D), jnp.float32),
        scratch_types=[
            pltpu.VMEM((b_per_w,), jnp.int32),
            pltpu.VMEM((b_per_w, D), jnp.float32),
            pltpu.SemaphoreType.DMA,
        ],
    )
    def gather_k(table_hbm, idx_hbm, out_hbm, idx_v, rows_v, sem):
        wid = lax.axis_index("s") * info.num_cores + lax.axis_index("c")
        base = wid * b_per_w
        pltpu.sync_copy(idx_hbm.at[pl.ds(base, b_per_w)], idx_v)
        pltpu.async_copy(table_hbm.at[idx_v], rows_v, sem).wait()
        pltpu.sync_copy(rows_v, out_hbm.at[pl.ds(base, b_per_w)])

    return gather_k


def _dot(a, b, dims):
    return lax.dot_general(a, b, (dims, ((), ())),
                           preferred_element_type=jnp.float32)


def _fused_body(loads_ref, h0_ref, ids_ref,
                hw1, hw2, b1w1, b1w2, b2w1, b2w2, tw1, tw2, emb_hbm,
                out_ref, wsc, embsc, sems):
    t = pl.program_id(0)

    @pl.when(t == 0)
    def _stage():
        es = []
        for r in range(4):
            bv = loads_ref[r, 0]
            bi = jnp.int32(0)
            for c in range(1, E):
                v = loads_ref[r, c]
                pred = v < bv
                bi = jnp.where(pred, jnp.int32(c), bi)
                bv = jnp.where(pred, v, bv)
            es.append(bi)
        w_hbms = (hw1, hw2, b1w1, b1w2, b2w1, b2w2, tw1, tw2)
        cps = []
        for i, wh in enumerate(w_hbms):
            e = es[i // 2]
            cp = pltpu.make_async_copy(wh.at[e], wsc.at[i], sems.at[i])
            cp.start()
            cps.append(cp)
        cpe = pltpu.make_async_copy(emb_hbm, embsc, sems.at[8])
        cpe.start()
        for cp in cps:
            cp.wait()
        cpe.wait()

    h = h0_ref[...]
    for i in range(4):
        a = jnp.maximum(_dot(h, wsc[2 * i], ((1,), (0,))), 0.0)
        h = _dot(a, wsc[2 * i + 1], ((1,), (0,)))
    logits = _dot(h, embsc[...], ((1,), (1,)))
    m = jnp.max(logits, axis=1, keepdims=True)
    lse = m + jnp.log(jnp.sum(jnp.exp(logits - m), axis=1, keepdims=True))
    ids_col = ids_ref[0]
    col = lax.broadcasted_iota(jnp.int32, (TILE, V), 1)
    correct = jnp.sum(jnp.where(col == ids_col, logits, 0.0), axis=1,
                      keepdims=True)
    part = jnp.sum(lse - correct, axis=0, keepdims=True) * (1.0 / N_TOK)

    @pl.when(t == 0)
    def _init():
        out_ref[...] = part

    @pl.when(t != 0)
    def _acc():
        out_ref[...] += part


@functools.lru_cache(maxsize=None)
def _fused_fn():
    wspec = pl.BlockSpec(memory_space=pltpu.MemorySpace.ANY)
    return pl.pallas_call(
        _fused_body,
        grid=(N_TILES,),
        in_specs=[
            pl.BlockSpec(memory_space=pltpu.SMEM),
            pl.BlockSpec((TILE, D), lambda t: (t, 0)),
            pl.BlockSpec((1, TILE, 1), lambda t: (t, 0, 0)),
            wspec, wspec, wspec, wspec, wspec, wspec, wspec, wspec,
            wspec,
        ],
        out_specs=pl.BlockSpec((1, 1), lambda t: (0, 0)),
        out_shape=jax.ShapeDtypeStruct((1, 1), jnp.float32),
        scratch_shapes=[
            pltpu.VMEM((8, D, F), jnp.float32),
            pltpu.VMEM((V, D), jnp.float32),
            pltpu.SemaphoreType.DMA((9,)),
        ],
        compiler_params=pltpu.CompilerParams(
            dimension_semantics=("arbitrary",),
        ),
    )


def kernel(input_ids, loads, emb, head_w1, head_w2, body1_w1, body1_w2,
           body2_w1, body2_w2, tail_w1, tail_w2):
    ids = input_ids.reshape(-1)
    hidden0 = _sc_gather_fn()(emb, ids)
    ids3 = ids.reshape(N_TILES, TILE, 1)
    out = _fused_fn()(loads, hidden0, ids3, head_w1, head_w2, body1_w1,
                      body1_w2, body2_w1, body2_w2, tail_w1, tail_w2, emb)
    return out[0, 0]


# speedup vs baseline: 1.5834x; 1.5834x over previous
import functools

import jax
import jax.numpy as jnp
from jax import lax
from jax.experimental import pallas as pl
from jax.experimental.pallas import tpu as pltpu
from jax.experimental.pallas import tpu_sc as plsc

B = 1
S = 2048
D = 1024
F = 1024
E = 8
V = 4096
N_TOK = B * S
TILE = 256
N_TILES = N_TOK // TILE


@functools.lru_cache(maxsize=None)
def _sc_gather_fn():
    info = plsc.get_sparse_core_info()
    nw = info.num_cores * info.num_subcores
    b_per_w = N_TOK // nw
    mesh = plsc.VectorSubcoreMesh(core_axis_name="c", subcore_axis_name="s")

    @functools.partial(
        pl.kernel,
        mesh=mesh,
        out_type=jax.ShapeDtypeStruct((N_TOK, D), jnp.float32),
        scratch_types=[
            pltpu.VMEM((b_per_w,), jnp.int32),
            pltpu.VMEM((b_per_w, D), jnp.float32),
            pltpu.SemaphoreType.DMA,
        ],
    )
    def gather_k(table_hbm, idx_hbm, out_hbm, idx_v, rows_v, sem):
        wid = lax.axis_index("s") * info.num_cores + lax.axis_index("c")
        base = wid * b_per_w
        pltpu.sync_copy(idx_hbm.at[pl.ds(base, b_per_w)], idx_v)
        pltpu.async_copy(table_hbm.at[idx_v], rows_v, sem).wait()
        pltpu.sync_copy(rows_v, out_hbm.at[pl.ds(base, b_per_w)])

    return gather_k


def _dot(a, b, dims):
    return lax.dot_general(a, b, (dims, ((), ())),
                           preferred_element_type=jnp.float32)


def _fused_body(loads_ref, h0_ref, ids_ref,
                hw1, hw2, b1w1, b1w2, b2w1, b2w2, tw1, tw2, emb_hbm,
                out_ref, wsc, embsc, sems):
    t = pl.program_id(0)

    @pl.when(t == 0)
    def _stage():
        es = []
        for r in range(4):
            bv = loads_ref[r, 0]
            bi = jnp.int32(0)
            for c in range(1, E):
                v = loads_ref[r, c]
                pred = v < bv
                bi = jnp.where(pred, jnp.int32(c), bi)
                bv = jnp.where(pred, v, bv)
            es.append(bi)
        w_hbms = (hw1, hw2, b1w1, b1w2, b2w1, b2w2, tw1, tw2)
        cps = []
        for i, wh in enumerate(w_hbms):
            e = es[i // 2]
            cp = pltpu.make_async_copy(wh.at[e], wsc.at[i], sems.at[i])
            cp.start()
            cps.append(cp)
        cpe = pltpu.make_async_copy(emb_hbm, embsc, sems.at[8])
        cpe.start()
        for cp in cps:
            cp.wait()
        cpe.wait()

    h = h0_ref[...]
    for i in range(4):
        a = jnp.maximum(_dot(h, wsc[2 * i], ((1,), (0,))), 0.0)
        h = _dot(a, wsc[2 * i + 1], ((1,), (0,)))
    logits = _dot(h, embsc[...], ((1,), (1,)))
    m = jnp.max(logits, axis=1, keepdims=True)
    lse = m + jnp.log(jnp.sum(jnp.exp(logits - m), axis=1, keepdims=True))
    ids_col = ids_ref[0]
    col = lax.broadcasted_iota(jnp.int32, (TILE, V), 1)
    correct = jnp.sum(jnp.where(col == ids_col, logits, 0.0), axis=1,
                      keepdims=True)
    part = jnp.sum(lse - correct, axis=0, keepdims=True) * (1.0 / N_TOK)

    @pl.when(t == 0)
    def _init():
        out_ref[...] = part

    @pl.when(t != 0)
    def _acc():
        out_ref[...] += part


@functools.lru_cache(maxsize=None)
def _fused_fn():
    wspec = pl.BlockSpec(memory_space=pl.ANY)
    return pl.pallas_call(
        _fused_body,
        grid=(N_TILES,),
        in_specs=[
            pl.BlockSpec(memory_space=pltpu.SMEM),
            pl.BlockSpec((TILE, D), lambda t: (t, 0)),
            pl.BlockSpec((1, TILE, 1), lambda t: (t, 0, 0)),
            wspec, wspec, wspec, wspec, wspec, wspec, wspec, wspec,
            wspec,
        ],
        out_specs=pl.BlockSpec((1, 1), lambda t: (0, 0)),
        out_shape=jax.ShapeDtypeStruct((1, 1), jnp.float32),
        scratch_shapes=[
            pltpu.VMEM((8, D, F), jnp.float32),
            pltpu.VMEM((V, D), jnp.float32),
            pltpu.SemaphoreType.DMA((9,)),
        ],
        compiler_params=pltpu.CompilerParams(
            dimension_semantics=("arbitrary",),
        ),
    )


def kernel(input_ids, loads, emb, head_w1, head_w2, body1_w1, body1_w2,
           body2_w1, body2_w2, tail_w1, tail_w2):
    ids = input_ids.reshape(-1)
    hidden0 = _sc_gather_fn()(emb, ids)
    ids3 = ids.reshape(N_TILES, TILE, 1)
    out = _fused_fn()(loads, hidden0, ids3, head_w1, head_w2, body1_w1,
                      body1_w2, body2_w1, body2_w2, tail_w1, tail_w2, emb)
    return out[0, 0]
